# baseline (speedup 1.0000x reference)
import functools

import jax
import jax.numpy as jnp
from jax import lax
from jax.experimental import pallas as pl
from jax.experimental.pallas import tpu as pltpu
from jax.experimental.pallas import tpu_sc as plsc

B = 16384
D = 32
NC = 2
NS = 16
L = 16
NW = NC * NS
BPW = B // NW
CHUNK = 128
NCH = BPW // CHUNK


@functools.lru_cache(maxsize=None)
def _make_sc_gather():
    mesh = plsc.VectorSubcoreMesh(core_axis_name="c", subcore_axis_name="s")

    @functools.partial(
        pl.kernel,
        mesh=mesh,
        compiler_params=pltpu.CompilerParams(use_tc_tiling_on_sc=False),
        out_type=[
            jax.ShapeDtypeStruct((B, D), jnp.float32),
            jax.ShapeDtypeStruct((B, D), jnp.float32),
            jax.ShapeDtypeStruct((B,), jnp.float32),
        ],
        scratch_types=[
            pltpu.VMEM((NCH, CHUNK), jnp.int32),
            pltpu.VMEM((NCH, CHUNK), jnp.int32),
            pltpu.VMEM((BPW, D), jnp.float32),
            pltpu.VMEM((BPW, D), jnp.float32),
            pltpu.VMEM((BPW,), jnp.float32),
            pltpu.VMEM((BPW,), jnp.float32),
            pltpu.VMEM((BPW,), jnp.float32),
            pltpu.SemaphoreType.DMA,
        ],
    )
    def sc_gather(uid_hbm, iid_hbm, uemb_hbm, iemb_hbm, ulin_hbm, ilin_hbm,
                  ue_out, ie_out, lin_out,
                  uidx_v, iidx_v, ue_v, ie_v, ul_v, il_v, lin_v, sem):
        wid = lax.axis_index("s") * NC + lax.axis_index("c")
        base = wid * BPW
        pltpu.sync_copy(uid_hbm.at[pl.ds(wid * NCH, NCH)], uidx_v)
        pltpu.sync_copy(iid_hbm.at[pl.ds(wid * NCH, NCH)], iidx_v)
        copies = []
        for j in range(NCH):
            dst = pl.ds(j * CHUNK, CHUNK)
            copies.append(pltpu.async_copy(
                uemb_hbm.at[uidx_v.at[j]], ue_v.at[dst], sem))
            copies.append(pltpu.async_copy(
                iemb_hbm.at[iidx_v.at[j]], ie_v.at[dst], sem))
            copies.append(pltpu.async_copy(
                ulin_hbm.at[uidx_v.at[j]], ul_v.at[dst], sem))
            copies.append(pltpu.async_copy(
                ilin_hbm.at[iidx_v.at[j]], il_v.at[dst], sem))
        for c in copies:
            c.wait()
        for j in range(BPW // L):
            sl = pl.ds(j * L, L)
            lin_v[sl] = ul_v[sl] + il_v[sl]
        pltpu.sync_copy(ue_v, ue_out.at[pl.ds(base, BPW)])
        pltpu.sync_copy(ie_v, ie_out.at[pl.ds(base, BPW)])
        pltpu.sync_copy(lin_v, lin_out.at[pl.ds(base, BPW)])

    return sc_gather


def _sc_gather(*args):
    return _make_sc_gather()(*args)


def _dense_body(ue_ref, ie_ref, lin_ref, w0ut_ref, w0it_ref, b0_ref,
                w1t_ref, b1_ref, woutt_ref, c_ref, out_ref):
    ue = ue_ref[...]
    ie = ie_ref[...]
    dn = (((1,), (1,)), ((), ()))
    h0 = lax.dot_general(w0ut_ref[...], ue, dn,
                         preferred_element_type=jnp.float32)
    h0 = h0 + lax.dot_general(w0it_ref[...], ie, dn,
                              preferred_element_type=jnp.float32)
    h0 = jnp.maximum(h0 + b0_ref[...], 0.0)
    dn2 = (((1,), (0,)), ((), ()))
    h1 = lax.dot_general(w1t_ref[...], h0, dn2,
                         preferred_element_type=jnp.float32)
    h1 = jnp.maximum(h1 + b1_ref[...], 0.0)
    dnn = lax.dot_general(woutt_ref[...], h1, dn2,
                          preferred_element_type=jnp.float32)
    fm = lax.dot_general(jnp.ones((1, D), jnp.float32), ue * ie, dn,
                         preferred_element_type=jnp.float32)
    logit = lin_ref[...][None, :] + fm + dnn + c_ref[0]
    out_ref[...] = (1.0 / (1.0 + jnp.exp(-logit)))[0]


def _dense(ue, ie, lin, w0ut, w0it, b0c, w1t, b1c, woutt, c):
    BB = 2048
    grid = (B // BB,)
    return pl.pallas_call(
        _dense_body,
        grid=grid,
        in_specs=[
            pl.BlockSpec((BB, D), lambda i: (i, 0)),
            pl.BlockSpec((BB, D), lambda i: (i, 0)),
            pl.BlockSpec((BB,), lambda i: (i,)),
            pl.BlockSpec((D, D), lambda i: (0, 0)),
            pl.BlockSpec((D, D), lambda i: (0, 0)),
            pl.BlockSpec((D, 1), lambda i: (0, 0)),
            pl.BlockSpec((D, D), lambda i: (0, 0)),
            pl.BlockSpec((D, 1), lambda i: (0, 0)),
            pl.BlockSpec((1, D), lambda i: (0, 0)),
            pl.BlockSpec(memory_space=pltpu.SMEM),
        ],
        out_specs=pl.BlockSpec((BB,), lambda i: (i,)),
        out_shape=jax.ShapeDtypeStruct((B,), jnp.float32),
    )(ue, ie, lin, w0ut, w0it, b0c, w1t, b1c, woutt, c)


def kernel(user_ids, item_ids, user_embedding, item_embedding, user_linear,
           item_linear, W0, b0, g0, beta0, W1, b1, g1, beta1, W_out, b_out,
           bias):
    eps = 1e-5
    s = 1.0 / jnp.sqrt(1.0 + eps)
    s0 = g0 * s
    s1 = g1 * s
    W0f = W0 * s0[None, :]
    b0f = b0 * s0 + beta0
    W1f = W1 * s1[None, :]
    b1f = b1 * s1 + beta1
    w0ut = jnp.transpose(W0f[:D])
    w0it = jnp.transpose(W0f[D:])
    w1t = jnp.transpose(W1f)
    woutt = jnp.transpose(W_out)
    c = (b_out + bias).reshape((1,))

    ue, ie, lin = _sc_gather(
        user_ids.astype(jnp.int32).reshape((NW * NCH, CHUNK)),
        item_ids.astype(jnp.int32).reshape((NW * NCH, CHUNK)),
        user_embedding, item_embedding,
        user_linear.reshape((user_linear.shape[0],)),
        item_linear.reshape((item_linear.shape[0],)))

    return _dense(ue, ie, lin, w0ut, w0it, b0f.reshape((D, 1)), w1t,
                  b1f.reshape((D, 1)), woutt, c)
